# Initial kernel scaffold; baseline (speedup 1.0000x reference)
import jax
import jax.numpy as jnp
from jax.experimental import pallas as pl


def kernel(mel_frame, W_in, b_in, cb0, cb1, W_out, b_out):
    raise NotImplementedError("write your pallas kernel here")


# speedup vs baseline: 1.1974x; 1.1974x over previous
import jax
import jax.numpy as jnp
from jax.experimental import pallas as pl

_BLK = 1024


def _rowsum64(v):
    acc = v[:, 0:8]
    for a in range(1, 8):
        acc = acc + v[:, 8 * a:8 * a + 8]
    acc = acc[:, 0:4] + acc[:, 4:8]
    acc = acc[:, 0:2] + acc[:, 2:4]
    return acc[:, 0:1] + acc[:, 1:2]


def _rvq_body(x_ref, w_in_ref, b_in_ref, cb0_ref, cb1_ref, w_out_ref,
              b_out_ref, o_ref):
    x = x_ref[...]
    z = jnp.dot(x, w_in_ref[...], preferred_element_type=jnp.float32)
    z = z + b_in_ref[...]

    def nearest_code(r, cb):
        r2 = _rowsum64(r * r)
        c2 = _rowsum64(cb * cb).T
        dist = r2 - 2.0 * jnp.dot(r, cb.T, preferred_element_type=jnp.float32)
        dist = dist + c2
        ind = jnp.argmin(dist, axis=1)
        onehot = (jax.lax.broadcasted_iota(jnp.int32, dist.shape, 1)
                  == ind[:, None]).astype(jnp.bfloat16)
        hi = cb.astype(jnp.bfloat16)
        rem = cb - hi.astype(jnp.float32)
        mid = rem.astype(jnp.bfloat16)
        lo = (rem - mid.astype(jnp.float32)).astype(jnp.bfloat16)
        sel = lambda limb: jnp.dot(onehot, limb,
                                   preferred_element_type=jnp.float32)
        return (sel(hi) + sel(mid)) + sel(lo)

    code0 = nearest_code(z, cb0_ref[...])
    code1 = nearest_code(z - code0, cb1_ref[...])
    zq = code0 + code1
    out = jnp.dot(zq, w_out_ref[...], preferred_element_type=jnp.float32)
    o_ref[...] = out + b_out_ref[...]


def kernel(mel_frame, W_in, b_in, cb0, cb1, W_out, b_out):
    b, t, d_in = mel_frame.shape
    n = b * t
    d = W_in.shape[1]
    k = cb0.shape[0]
    x = mel_frame.reshape(n, d_in)
    out = pl.pallas_call(
        _rvq_body,
        grid=(n // _BLK,),
        in_specs=[
            pl.BlockSpec((_BLK, d_in), lambda i: (i, 0)),
            pl.BlockSpec((d_in, d), lambda i: (0, 0)),
            pl.BlockSpec((1, d), lambda i: (0, 0)),
            pl.BlockSpec((k, d), lambda i: (0, 0)),
            pl.BlockSpec((k, d), lambda i: (0, 0)),
            pl.BlockSpec((d, d_in), lambda i: (0, 0)),
            pl.BlockSpec((1, d_in), lambda i: (0, 0)),
        ],
        out_specs=pl.BlockSpec((_BLK, d_in), lambda i: (i, 0)),
        out_shape=jax.ShapeDtypeStruct((n, d_in), jnp.float32),
    )(x, W_in, b_in.reshape(1, d), cb0, cb1, W_out, b_out.reshape(1, d_in))
    return out.reshape(b, t, d_in)


# speedup vs baseline: 3.5058x; 2.9277x over previous
import jax
import jax.numpy as jnp
from jax.experimental import pallas as pl

_BLK = 1024


def _rowsum64_sublane(sT):
    acc = sT[0:8, :]
    for a in range(1, 8):
        acc = acc + sT[8 * a:8 * a + 8, :]
    acc = acc[0:4, :] + acc[4:8, :]
    acc = acc[0:2, :] + acc[2:4, :]
    return acc[0:1, :] + acc[1:2, :]


def _rvq_body(xT_ref, w_inT_ref, b_inT_ref, cb0_ref, cb1_ref,
              cb0T_ref, cb1T_ref,
              c20_ref, c21_ref, w_outT_ref, b_outT_ref, o_ref):
    xT = xT_ref[...]
    zT = jnp.dot(w_inT_ref[...], xT, preferred_element_type=jnp.float32)
    zT = zT + b_inT_ref[...]

    def nearest_code(rT, cb, cbT, c2):
        r2 = _rowsum64_sublane(rT * rT)
        distT = r2 - 2.0 * jnp.dot(cb, rT, preferred_element_type=jnp.float32)
        distT = distT + c2
        ind = jnp.argmin(distT, axis=0)
        onehotT = (jax.lax.broadcasted_iota(jnp.int32, distT.shape, 0)
                   == ind[None, :]).astype(jnp.bfloat16)
        hi = cbT.astype(jnp.bfloat16)
        rem = cbT - hi.astype(jnp.float32)
        mid = rem.astype(jnp.bfloat16)
        lo = (rem - mid.astype(jnp.float32)).astype(jnp.bfloat16)
        sel = lambda limb: jnp.dot(limb, onehotT,
                                   preferred_element_type=jnp.float32)
        return (sel(hi) + sel(mid)) + sel(lo)

    code0T = nearest_code(zT, cb0_ref[...], cb0T_ref[...], c20_ref[...])
    code1T = nearest_code(zT - code0T, cb1_ref[...], cb1T_ref[...],
                          c21_ref[...])
    zqT = code0T + code1T
    outT = jnp.dot(w_outT_ref[...], zqT, preferred_element_type=jnp.float32)
    o_ref[...] = outT + b_outT_ref[...]


def _codebook_sqnorm(cb):
    s = cb * cb
    w = s.reshape(cb.shape[0], 8, 8)
    acc = w[:, 0, :]
    for a in range(1, 8):
        acc = acc + w[:, a, :]
    acc = acc[:, 0:4] + acc[:, 4:8]
    acc = acc[:, 0:2] + acc[:, 2:4]
    return acc[:, 0:1] + acc[:, 1:2]


def kernel(mel_frame, W_in, b_in, cb0, cb1, W_out, b_out):
    b, t, d_in = mel_frame.shape
    n = b * t
    d = W_in.shape[1]
    k = cb0.shape[0]
    xT = mel_frame.reshape(n, d_in).T
    c20 = _codebook_sqnorm(cb0)
    c21 = _codebook_sqnorm(cb1)
    full = lambda shape: pl.BlockSpec(shape, lambda i: (0, 0))
    outT = pl.pallas_call(
        _rvq_body,
        grid=(n // _BLK,),
        in_specs=[
            pl.BlockSpec((d_in, _BLK), lambda i: (0, i)),
            full((d, d_in)),
            full((d, 1)),
            full((k, d)),
            full((k, d)),
            full((d, k)),
            full((d, k)),
            full((k, 1)),
            full((k, 1)),
            full((d_in, d)),
            full((d_in, 1)),
        ],
        out_specs=pl.BlockSpec((d_in, _BLK), lambda i: (0, i)),
        out_shape=jax.ShapeDtypeStruct((d_in, n), jnp.float32),
    )(xT, W_in.T, b_in.reshape(d, 1), cb0, cb1, cb0.T, cb1.T,
      c20, c21, W_out.T, b_out.reshape(d_in, 1))
    return outT.T.reshape(b, t, d_in)


# speedup vs baseline: 5.1566x; 1.4709x over previous
import jax
import jax.numpy as jnp
from jax.experimental import pallas as pl

_BLK = 1024


def _rowsum64_sublane(sT):
    acc = sT[0:8, :]
    for a in range(1, 8):
        acc = acc + sT[8 * a:8 * a + 8, :]
    acc = acc[0:4, :] + acc[4:8, :]
    acc = acc[0:2, :] + acc[2:4, :]
    return acc[0:1, :] + acc[1:2, :]


def _rvq_body(x_ref, w_in_ref, b_in_ref, cb0_ref, cb1_ref,
              cb0T_ref, cb1T_ref, c20_ref, c21_ref,
              w_out_ref, b_out_ref, o_ref):
    x = x_ref[...]
    z = jnp.dot(x, w_in_ref[...], preferred_element_type=jnp.float32)
    z = z + b_in_ref[...]
    zT = z.T

    def nearest_code(rT, cb, cbT, c2):
        r2 = _rowsum64_sublane(rT * rT)
        distT = r2 - 2.0 * jnp.dot(cb, rT, preferred_element_type=jnp.float32)
        distT = distT + c2
        ind = jnp.argmin(distT, axis=0)
        onehotT = (jax.lax.broadcasted_iota(jnp.int32, distT.shape, 0)
                   == ind[None, :]).astype(jnp.bfloat16)
        hi = cbT.astype(jnp.bfloat16)
        rem = cbT - hi.astype(jnp.float32)
        mid = rem.astype(jnp.bfloat16)
        lo = (rem - mid.astype(jnp.float32)).astype(jnp.bfloat16)
        sel = lambda limb: jnp.dot(limb, onehotT,
                                   preferred_element_type=jnp.float32)
        return (sel(hi) + sel(mid)) + sel(lo)

    code0T = nearest_code(zT, cb0_ref[...], cb0T_ref[...], c20_ref[...])
    code1T = nearest_code(zT - code0T, cb1_ref[...], cb1T_ref[...],
                          c21_ref[...])
    zq = (code0T + code1T).T
    out = jnp.dot(zq, w_out_ref[...], preferred_element_type=jnp.float32)
    o_ref[...] = out + b_out_ref[...]


def _codebook_sqnorm(cb):
    s = cb * cb
    w = s.reshape(cb.shape[0], 8, 8)
    acc = w[:, 0, :]
    for a in range(1, 8):
        acc = acc + w[:, a, :]
    acc = acc[:, 0:4] + acc[:, 4:8]
    acc = acc[:, 0:2] + acc[:, 2:4]
    return acc[:, 0:1] + acc[:, 1:2]


def kernel(mel_frame, W_in, b_in, cb0, cb1, W_out, b_out):
    b, t, d_in = mel_frame.shape
    n = b * t
    d = W_in.shape[1]
    k = cb0.shape[0]
    x = mel_frame.reshape(n, d_in)
    c20 = _codebook_sqnorm(cb0)
    c21 = _codebook_sqnorm(cb1)
    full = lambda shape: pl.BlockSpec(shape, lambda i: (0, 0))
    out = pl.pallas_call(
        _rvq_body,
        grid=(n // _BLK,),
        in_specs=[
            pl.BlockSpec((_BLK, d_in), lambda i: (i, 0)),
            full((d_in, d)),
            full((1, d)),
            full((k, d)),
            full((k, d)),
            full((d, k)),
            full((d, k)),
            full((k, 1)),
            full((k, 1)),
            full((d, d_in)),
            full((1, d_in)),
        ],
        out_specs=pl.BlockSpec((_BLK, d_in), lambda i: (i, 0)),
        out_shape=jax.ShapeDtypeStruct((n, d_in), jnp.float32),
    )(x, W_in, b_in.reshape(1, d), cb0, cb1, cb0.T, cb1.T,
      c20, c21, W_out, b_out.reshape(1, d_in))
    return out.reshape(b, t, d_in)


# speedup vs baseline: 6.8795x; 1.3341x over previous
import jax
import jax.numpy as jnp
from jax.experimental import pallas as pl

_BLK = 2048


def _rowsum64_sublane(sT):
    acc = sT[0:8, :]
    for a in range(1, 8):
        acc = acc + sT[8 * a:8 * a + 8, :]
    acc = acc[0:4, :] + acc[4:8, :]
    acc = acc[0:2, :] + acc[2:4, :]
    return acc[0:1, :] + acc[1:2, :]


def _rvq_body(x_ref, w_in_ref, b_in_ref, cb0_ref, cb1_ref,
              cb0T_ref, cb1T_ref, c20_ref, c21_ref,
              w_out_ref, b_out_ref, o_ref):
    x = x_ref[...]
    z = jnp.dot(x, w_in_ref[...], preferred_element_type=jnp.float32)
    z = z + b_in_ref[...]
    zT = z.T

    def nearest_code(rT, cb, cbT, c2):
        r2 = _rowsum64_sublane(rT * rT)
        distT = r2 - 2.0 * jnp.dot(cb, rT, preferred_element_type=jnp.float32)
        distT = distT + c2
        ind = jnp.argmin(distT, axis=0)
        onehotT = (jax.lax.broadcasted_iota(jnp.int32, distT.shape, 0)
                   == ind[None, :]).astype(jnp.bfloat16)
        hi = cbT.astype(jnp.bfloat16)
        rem = cbT - hi.astype(jnp.float32)
        mid = rem.astype(jnp.bfloat16)
        lo = (rem - mid.astype(jnp.float32)).astype(jnp.bfloat16)
        sel = lambda limb: jnp.dot(limb, onehotT,
                                   preferred_element_type=jnp.float32)
        return (sel(hi) + sel(mid)) + sel(lo)

    code0T = nearest_code(zT, cb0_ref[...], cb0T_ref[...], c20_ref[...])
    code1T = nearest_code(zT - code0T, cb1_ref[...], cb1T_ref[...],
                          c21_ref[...])
    zq = (code0T + code1T).T
    out = jnp.dot(zq, w_out_ref[...], preferred_element_type=jnp.float32)
    o_ref[...] = out + b_out_ref[...]


def _codebook_sqnorm(cb):
    s = cb * cb
    w = s.reshape(cb.shape[0], 8, 8)
    acc = w[:, 0, :]
    for a in range(1, 8):
        acc = acc + w[:, a, :]
    acc = acc[:, 0:4] + acc[:, 4:8]
    acc = acc[:, 0:2] + acc[:, 2:4]
    return acc[:, 0:1] + acc[:, 1:2]


def kernel(mel_frame, W_in, b_in, cb0, cb1, W_out, b_out):
    b, t, d_in = mel_frame.shape
    n = b * t
    d = W_in.shape[1]
    k = cb0.shape[0]
    x = mel_frame.reshape(n, d_in)
    c20 = _codebook_sqnorm(cb0)
    c21 = _codebook_sqnorm(cb1)
    full = lambda shape: pl.BlockSpec(shape, lambda i: (0, 0))
    out = pl.pallas_call(
        _rvq_body,
        grid=(n // _BLK,),
        in_specs=[
            pl.BlockSpec((_BLK, d_in), lambda i: (i, 0)),
            full((d_in, d)),
            full((1, d)),
            full((k, d)),
            full((k, d)),
            full((d, k)),
            full((d, k)),
            full((k, 1)),
            full((k, 1)),
            full((d, d_in)),
            full((1, d_in)),
        ],
        out_specs=pl.BlockSpec((_BLK, d_in), lambda i: (i, 0)),
        out_shape=jax.ShapeDtypeStruct((n, d_in), jnp.float32),
    )(x, W_in, b_in.reshape(1, d), cb0, cb1, cb0.T, cb1.T,
      c20, c21, W_out, b_out.reshape(1, d_in))
    return out.reshape(b, t, d_in)


# speedup vs baseline: 8.4247x; 1.2246x over previous
import jax
import jax.numpy as jnp
from jax.experimental import pallas as pl

_BLK = 4096


def _rowsum64_sublane(sT):
    acc = sT[0:8, :]
    for a in range(1, 8):
        acc = acc + sT[8 * a:8 * a + 8, :]
    acc = acc[0:4, :] + acc[4:8, :]
    acc = acc[0:2, :] + acc[2:4, :]
    return acc[0:1, :] + acc[1:2, :]


def _rvq_body(x_ref, w_in_ref, b_in_ref, cb0_ref, cb1_ref,
              cb0T_ref, cb1T_ref, c20_ref, c21_ref,
              w_out_ref, b_out_ref, o_ref):
    x = x_ref[...]
    z = jnp.dot(x, w_in_ref[...], preferred_element_type=jnp.float32)
    z = z + b_in_ref[...]
    zT = z.T

    def nearest_code(rT, cb, cbT, c2):
        r2 = _rowsum64_sublane(rT * rT)
        distT = r2 - 2.0 * jnp.dot(cb, rT, preferred_element_type=jnp.float32)
        distT = distT + c2
        ind = jnp.argmin(distT, axis=0)
        onehotT = (jax.lax.broadcasted_iota(jnp.int32, distT.shape, 0)
                   == ind[None, :]).astype(jnp.bfloat16)
        hi = cbT.astype(jnp.bfloat16)
        rem = cbT - hi.astype(jnp.float32)
        mid = rem.astype(jnp.bfloat16)
        lo = (rem - mid.astype(jnp.float32)).astype(jnp.bfloat16)
        sel = lambda limb: jnp.dot(limb, onehotT,
                                   preferred_element_type=jnp.float32)
        return (sel(hi) + sel(mid)) + sel(lo)

    code0T = nearest_code(zT, cb0_ref[...], cb0T_ref[...], c20_ref[...])
    code1T = nearest_code(zT - code0T, cb1_ref[...], cb1T_ref[...],
                          c21_ref[...])
    zq = (code0T + code1T).T
    out = jnp.dot(zq, w_out_ref[...], preferred_element_type=jnp.float32)
    o_ref[...] = out + b_out_ref[...]


def _codebook_sqnorm(cb):
    s = cb * cb
    w = s.reshape(cb.shape[0], 8, 8)
    acc = w[:, 0, :]
    for a in range(1, 8):
        acc = acc + w[:, a, :]
    acc = acc[:, 0:4] + acc[:, 4:8]
    acc = acc[:, 0:2] + acc[:, 2:4]
    return acc[:, 0:1] + acc[:, 1:2]


def kernel(mel_frame, W_in, b_in, cb0, cb1, W_out, b_out):
    b, t, d_in = mel_frame.shape
    n = b * t
    d = W_in.shape[1]
    k = cb0.shape[0]
    x = mel_frame.reshape(n, d_in)
    c20 = _codebook_sqnorm(cb0)
    c21 = _codebook_sqnorm(cb1)
    full = lambda shape: pl.BlockSpec(shape, lambda i: (0, 0))
    out = pl.pallas_call(
        _rvq_body,
        grid=(n // _BLK,),
        in_specs=[
            pl.BlockSpec((_BLK, d_in), lambda i: (i, 0)),
            full((d_in, d)),
            full((1, d)),
            full((k, d)),
            full((k, d)),
            full((d, k)),
            full((d, k)),
            full((k, 1)),
            full((k, 1)),
            full((d, d_in)),
            full((1, d_in)),
        ],
        out_specs=pl.BlockSpec((_BLK, d_in), lambda i: (i, 0)),
        out_shape=jax.ShapeDtypeStruct((n, d_in), jnp.float32),
    )(x, W_in, b_in.reshape(1, d), cb0, cb1, cb0.T, cb1.T,
      c20, c21, W_out, b_out.reshape(1, d_in))
    return out.reshape(b, t, d_in)


# speedup vs baseline: 9.0701x; 1.0766x over previous
import jax
import jax.numpy as jnp
from jax.experimental import pallas as pl

_BLK = 8192


def _rowsum64_sublane(sT):
    acc = sT[0:8, :]
    for a in range(1, 8):
        acc = acc + sT[8 * a:8 * a + 8, :]
    acc = acc[0:4, :] + acc[4:8, :]
    acc = acc[0:2, :] + acc[2:4, :]
    return acc[0:1, :] + acc[1:2, :]


def _rvq_body(x_ref, w_in_ref, b_in_ref, cb0_ref, cb1_ref,
              cb0T_ref, cb1T_ref, c20_ref, c21_ref,
              w_out_ref, b_out_ref, o_ref):
    x = x_ref[...]
    z = jnp.dot(x, w_in_ref[...], preferred_element_type=jnp.float32)
    z = z + b_in_ref[...]
    zT = z.T

    def nearest_code(rT, cb, cbT, c2):
        r2 = _rowsum64_sublane(rT * rT)
        distT = r2 - 2.0 * jnp.dot(cb, rT, preferred_element_type=jnp.float32)
        distT = distT + c2
        ind = jnp.argmin(distT, axis=0)
        onehotT = (jax.lax.broadcasted_iota(jnp.int32, distT.shape, 0)
                   == ind[None, :]).astype(jnp.bfloat16)
        hi = cbT.astype(jnp.bfloat16)
        rem = cbT - hi.astype(jnp.float32)
        mid = rem.astype(jnp.bfloat16)
        lo = (rem - mid.astype(jnp.float32)).astype(jnp.bfloat16)
        sel = lambda limb: jnp.dot(limb, onehotT,
                                   preferred_element_type=jnp.float32)
        return (sel(hi) + sel(mid)) + sel(lo)

    code0T = nearest_code(zT, cb0_ref[...], cb0T_ref[...], c20_ref[...])
    code1T = nearest_code(zT - code0T, cb1_ref[...], cb1T_ref[...],
                          c21_ref[...])
    zq = (code0T + code1T).T
    out = jnp.dot(zq, w_out_ref[...], preferred_element_type=jnp.float32)
    o_ref[...] = out + b_out_ref[...]


def _codebook_sqnorm(cb):
    s = cb * cb
    w = s.reshape(cb.shape[0], 8, 8)
    acc = w[:, 0, :]
    for a in range(1, 8):
        acc = acc + w[:, a, :]
    acc = acc[:, 0:4] + acc[:, 4:8]
    acc = acc[:, 0:2] + acc[:, 2:4]
    return acc[:, 0:1] + acc[:, 1:2]


def kernel(mel_frame, W_in, b_in, cb0, cb1, W_out, b_out):
    b, t, d_in = mel_frame.shape
    n = b * t
    d = W_in.shape[1]
    k = cb0.shape[0]
    x = mel_frame.reshape(n, d_in)
    c20 = _codebook_sqnorm(cb0)
    c21 = _codebook_sqnorm(cb1)
    full = lambda shape: pl.BlockSpec(shape, lambda i: (0, 0))
    out = pl.pallas_call(
        _rvq_body,
        grid=(n // _BLK,),
        in_specs=[
            pl.BlockSpec((_BLK, d_in), lambda i: (i, 0)),
            full((d_in, d)),
            full((1, d)),
            full((k, d)),
            full((k, d)),
            full((d, k)),
            full((d, k)),
            full((k, 1)),
            full((k, 1)),
            full((d, d_in)),
            full((1, d_in)),
        ],
        out_specs=pl.BlockSpec((_BLK, d_in), lambda i: (i, 0)),
        out_shape=jax.ShapeDtypeStruct((n, d_in), jnp.float32),
    )(x, W_in, b_in.reshape(1, d), cb0, cb1, cb0.T, cb1.T,
      c20, c21, W_out, b_out.reshape(1, d_in))
    return out.reshape(b, t, d_in)
